# Initial kernel scaffold; baseline (speedup 1.0000x reference)
import jax
import jax.numpy as jnp
from jax.experimental import pallas as pl


def kernel(z_a, z_b, z_c, W1, b1, W2, b2, perm_a, perm_b, perm_c):
    raise NotImplementedError("write your pallas kernel here")


# speedup vs baseline: 11.2713x; 11.2713x over previous
import functools

import jax
import jax.numpy as jnp
from jax import lax
from jax.experimental import pallas as pl
from jax.experimental.pallas import tpu as pltpu
from jax.experimental.pallas import tpu_sc as plsc

BATCH = 1_000_000
D = 16
HID = 16
OUT = 8
NA, NB, NC = 500_000, 300_000, 200_000

PACK = 8
RP = 4096
NW = 32
CH = 4096


def _mlp_body(zp_ref, w1_ref, b1_ref, w2_ref, b2_ref, out_ref):
    h = jnp.dot(zp_ref[...], w1_ref[...], preferred_element_type=jnp.float32)
    h = jnp.maximum(h + b1_ref[...], 0.0)
    o = jnp.dot(h, w2_ref[...], preferred_element_type=jnp.float32)
    out_ref[...] = o + b2_ref[...]


def _mlp(zp, w1b, b1b, w2b, b2b):
    npk = zp.shape[0]
    grid = (npk + RP - 1) // RP
    return pl.pallas_call(
        _mlp_body,
        grid=(grid,),
        in_specs=[
            pl.BlockSpec((RP, PACK * D), lambda i: (i, 0)),
            pl.BlockSpec((PACK * D, PACK * HID), lambda i: (0, 0)),
            pl.BlockSpec((1, PACK * HID), lambda i: (0, 0)),
            pl.BlockSpec((PACK * HID, PACK * OUT), lambda i: (0, 0)),
            pl.BlockSpec((1, PACK * OUT), lambda i: (0, 0)),
        ],
        out_specs=pl.BlockSpec((RP, PACK * OUT), lambda i: (i, 0)),
        out_shape=jax.ShapeDtypeStruct((npk, PACK * OUT), jnp.float32),
    )(zp, w1b, b1b, w2b, b2b)


def _scatter_body(ha, hb, hc, pa, pb, pc, out, idx_v, rows_v, sem):
    w = lax.axis_index("s") * 2 + lax.axis_index("c")

    for h_ref, p_ref, n in ((ha, pa, NA), (hb, pb, NB), (hc, pc, NC)):
        nchunks = -(-n // CH)
        last = n - CH

        def body(i, _, h_ref=h_ref, p_ref=p_ref, last=last):
            base = jnp.minimum((w + i * NW) * CH, last)
            pltpu.sync_copy(p_ref.at[pl.ds(base, CH)], idx_v)
            pltpu.sync_copy(h_ref.at[pl.ds(base, CH)], rows_v)
            pltpu.async_copy(rows_v, out.at[idx_v], sem).wait()
            return 0

        niter = (nchunks - w + NW - 1) // NW
        lax.fori_loop(0, niter, body, 0)


def _scatter(ha, hb, hc, pa, pb, pc):
    mesh = plsc.VectorSubcoreMesh(core_axis_name="c", subcore_axis_name="s")
    f = functools.partial(
        pl.kernel,
        mesh=mesh,
        compiler_params=pltpu.CompilerParams(use_tc_tiling_on_sc=False),
        out_type=jax.ShapeDtypeStruct((BATCH, OUT), jnp.float32),
        scratch_types=[
            pltpu.VMEM((CH,), jnp.int32),
            pltpu.VMEM((CH, OUT), jnp.float32),
            pltpu.SemaphoreType.DMA,
        ],
    )(_scatter_body)
    return f(ha, hb, hc, pa, pb, pc)


def kernel(z_a, z_b, z_c, W1, b1, W2, b2, perm_a, perm_b, perm_c):
    eye = jnp.eye(PACK, dtype=jnp.float32)
    w1b = jnp.kron(eye, W1)
    b1b = jnp.tile(b1, PACK)[None, :]
    w2b = jnp.kron(eye, W2)
    b2b = jnp.tile(b2, PACK)[None, :]

    outs = []
    for z, n in ((z_a, NA), (z_b, NB), (z_c, NC)):
        zp = z.reshape(n // PACK, PACK * D)
        hp = _mlp(zp, w1b, b1b, w2b, b2b)
        outs.append(hp.reshape(n, OUT))
    ha, hb, hc = outs

    pa = perm_a.astype(jnp.int32)
    pb = perm_b.astype(jnp.int32)
    pc = perm_c.astype(jnp.int32)
    return _scatter(ha, hb, hc, pa, pb, pc)


# speedup vs baseline: 11.7053x; 1.0385x over previous
import functools

import jax
import jax.numpy as jnp
from jax import lax
from jax.experimental import pallas as pl
from jax.experimental.pallas import tpu as pltpu
from jax.experimental.pallas import tpu_sc as plsc

BATCH = 1_000_000
D = 16
HID = 16
OUT = 8
NA, NB, NC = 500_000, 300_000, 200_000

PACK = 16
RP = 4096
NW = 32
CH = 4096


def _mlp_body(zp_ref, w1_ref, b1_ref, w2_ref, b2_ref, out_ref):
    h = jnp.dot(zp_ref[...], w1_ref[...], preferred_element_type=jnp.float32)
    h = jnp.maximum(h + b1_ref[...], 0.0)
    o = jnp.dot(h, w2_ref[...], preferred_element_type=jnp.float32)
    out_ref[...] = o + b2_ref[...]


def _mlp(zp, w1b, b1b, w2b, b2b):
    npk = zp.shape[0]
    grid = (npk + RP - 1) // RP
    return pl.pallas_call(
        _mlp_body,
        grid=(grid,),
        in_specs=[
            pl.BlockSpec((RP, PACK * D), lambda i: (i, 0)),
            pl.BlockSpec((PACK * D, PACK * HID), lambda i: (0, 0)),
            pl.BlockSpec((1, PACK * HID), lambda i: (0, 0)),
            pl.BlockSpec((PACK * HID, PACK * OUT), lambda i: (0, 0)),
            pl.BlockSpec((1, PACK * OUT), lambda i: (0, 0)),
        ],
        out_specs=pl.BlockSpec((RP, PACK * OUT), lambda i: (i, 0)),
        out_shape=jax.ShapeDtypeStruct((npk, PACK * OUT), jnp.float32),
    )(zp, w1b, b1b, w2b, b2b)


def _scatter_body(ha, hb, hc, pa, pb, pc, out, idx_v, rows_v, sem):
    w = lax.axis_index("s") * 2 + lax.axis_index("c")

    for h_ref, p_ref, n in ((ha, pa, NA), (hb, pb, NB), (hc, pc, NC)):
        nchunks = -(-n // CH)
        last = n - CH

        def body(i, _, h_ref=h_ref, p_ref=p_ref, last=last):
            base = jnp.minimum((w + i * NW) * CH, last)
            pltpu.sync_copy(p_ref.at[pl.ds(base, CH)], idx_v)
            pltpu.sync_copy(h_ref.at[pl.ds(base, CH)], rows_v)
            pltpu.async_copy(rows_v, out.at[idx_v], sem).wait()
            return 0

        niter = (nchunks - w + NW - 1) // NW
        lax.fori_loop(0, niter, body, 0)


def _scatter(ha, hb, hc, pa, pb, pc):
    mesh = plsc.VectorSubcoreMesh(core_axis_name="c", subcore_axis_name="s")
    f = functools.partial(
        pl.kernel,
        mesh=mesh,
        compiler_params=pltpu.CompilerParams(use_tc_tiling_on_sc=False),
        out_type=jax.ShapeDtypeStruct((BATCH, OUT), jnp.float32),
        scratch_types=[
            pltpu.VMEM((CH,), jnp.int32),
            pltpu.VMEM((CH, OUT), jnp.float32),
            pltpu.SemaphoreType.DMA,
        ],
    )(_scatter_body)
    return f(ha, hb, hc, pa, pb, pc)


def kernel(z_a, z_b, z_c, W1, b1, W2, b2, perm_a, perm_b, perm_c):
    eye = jnp.eye(PACK, dtype=jnp.float32)
    w1b = jnp.kron(eye, W1)
    b1b = jnp.tile(b1, PACK)[None, :]
    w2b = jnp.kron(eye, W2)
    b2b = jnp.tile(b2, PACK)[None, :]

    outs = []
    for z, n in ((z_a, NA), (z_b, NB), (z_c, NC)):
        zp = z.reshape(n // PACK, PACK * D)
        hp = _mlp(zp, w1b, b1b, w2b, b2b)
        outs.append(hp.reshape(n, OUT))
    ha, hb, hc = outs

    pa = perm_a.astype(jnp.int32)
    pb = perm_b.astype(jnp.int32)
    pc = perm_c.astype(jnp.int32)
    return _scatter(ha, hb, hc, pa, pb, pc)


# speedup vs baseline: 13.9330x; 1.1903x over previous
import functools

import jax
import jax.numpy as jnp
from jax import lax
from jax.experimental import pallas as pl
from jax.experimental.pallas import tpu as pltpu
from jax.experimental.pallas import tpu_sc as plsc

BATCH = 1_000_000
D = 16
HID = 16
OUT = 8
NA, NB, NC = 500_000, 300_000, 200_000

NW = 32
CH = 2048
TPC = CH // 128
CT = 32768

PAD_A = (-NA) % 128
PAD_B = (-NB) % 128
PAD_C = (-NC) % 128
NPA, NPB, NPC = NA + PAD_A, NB + PAD_B, NC + PAD_C
NOUT = ((BATCH + PAD_A + PAD_B + PAD_C + 127) // 128) * 128
OTILES = NOUT // 128


def _mlp_body(zt_ref, w1_ref, b1_ref, w2_ref, b2_ref, out_ref):
    zt = zt_ref[...]
    ht = lax.dot_general(w1_ref[...], zt, (((0,), (0,)), ((), ())),
                         preferred_element_type=jnp.float32)
    ht = jnp.maximum(ht + b1_ref[...][:, 0:1], 0.0)
    ot = lax.dot_general(w2_ref[...], ht, (((0,), (0,)), ((), ())),
                         preferred_element_type=jnp.float32)
    ot = ot + b2_ref[...][:, 0:1]
    out_ref[...] = ot.reshape(OUT, CT // 128, 128).transpose(1, 0, 2)


def _mlp(zt, b1t, b2t, w1, w2, npad):
    n = zt.shape[1]
    grid = (npad + CT - 1) // CT
    return pl.pallas_call(
        _mlp_body,
        grid=(grid,),
        in_specs=[
            pl.BlockSpec((D, CT), lambda i: (0, i)),
            pl.BlockSpec((D, HID), lambda i: (0, 0)),
            pl.BlockSpec((HID, 128), lambda i: (0, 0)),
            pl.BlockSpec((HID, OUT), lambda i: (0, 0)),
            pl.BlockSpec((OUT, 128), lambda i: (0, 0)),
        ],
        out_specs=pl.BlockSpec((CT // 128, OUT, 128), lambda i: (i, 0, 0)),
        out_shape=jax.ShapeDtypeStruct((npad // 128, OUT, 128), jnp.float32),
    )(zt, w1, b1t, w2, b2t)


def _scatter_body(ha, hb, hc, pa, pb, pc, out, idx_v, tiles_v, rows_v, sst,
                  sem):
    s_id = lax.axis_index("s")
    w = s_id * 2 + lax.axis_index("c")
    lanes = lax.iota(jnp.int32, 16)

    for h_ref, p_ref, n in ((ha, pa, NPA), (hb, pb, NPB), (hc, pc, NPC)):
        nchunks = n // CH if n % CH == 0 else n // CH + 1
        last = n - CH

        def chunk(i, _, h_ref=h_ref, p_ref=p_ref, last=last):
            base = jnp.minimum((w + i * NW) * CH, last)
            pltpu.sync_copy(p_ref.at[pl.ds(base // 128, TPC)], idx_v)
            pltpu.sync_copy(h_ref.at[pl.ds(base // 128, TPC)], tiles_v)
            for j in range(OUT):
                pltpu.sync_copy(tiles_v.at[:, j, :], sst.at[s_id, :, :, j])
            pltpu.sync_copy(sst.at[s_id], rows_v)
            copies = [
                pltpu.async_copy(rows_v.at[t], out.at[idx_v.at[t]], sem)
                for t in range(TPC)
            ]
            for c in copies:
                c.wait()
            return 0

        niter = (nchunks - w + NW - 1) // NW
        lax.fori_loop(0, niter, chunk, 0)


def _scatter(ha, hb, hc, pa, pb, pc):
    mesh = plsc.VectorSubcoreMesh(core_axis_name="c", subcore_axis_name="s")
    f = functools.partial(
        pl.kernel,
        mesh=mesh,
        compiler_params=pltpu.CompilerParams(use_tc_tiling_on_sc=False),
        out_type=jax.ShapeDtypeStruct((NOUT, OUT), jnp.float32),
        scratch_types=[
            pltpu.VMEM((TPC, 128), jnp.int32),
            pltpu.VMEM((TPC, OUT, 128), jnp.float32),
            pltpu.VMEM((TPC, 128, OUT), jnp.float32),
            pltpu.VMEM_SHARED((16, TPC, 128, OUT), jnp.float32),
            pltpu.SemaphoreType.DMA,
        ],
    )(_scatter_body)
    return f(ha, hb, hc, pa, pb, pc)


TB = 64


def _unpack_body(s_ref, out_ref):
    x = s_ref[...]
    y = jnp.transpose(x, (0, 2, 1))
    z = jnp.transpose(y, (1, 0, 2))
    out_ref[...] = z.reshape(OUT, TB * 128)


def _to_canonical(s):
    s3 = s.reshape(OTILES, 128, OUT)
    grid = (BATCH + TB * 128 - 1) // (TB * 128)
    return pl.pallas_call(
        _unpack_body,
        grid=(grid,),
        in_specs=[pl.BlockSpec((TB, 128, OUT), lambda i: (i, 0, 0))],
        out_specs=pl.BlockSpec((OUT, TB * 128), lambda i: (0, i)),
        out_shape=jax.ShapeDtypeStruct((OUT, BATCH), jnp.float32),
    )(s3)


def kernel(z_a, z_b, z_c, W1, b1, W2, b2, perm_a, perm_b, perm_c):
    b1t = jnp.tile(b1[:, None], (1, 128))
    b2t = jnp.tile(b2[:, None], (1, 128))

    ha = _mlp(z_a.T, b1t, b2t, W1, W2, NPA)
    hb = _mlp(z_b.T, b1t, b2t, W1, W2, NPB)
    hc = _mlp(z_c.T, b1t, b2t, W1, W2, NPC)

    pa = jnp.concatenate(
        [perm_a.astype(jnp.int32), BATCH + jnp.arange(PAD_A, dtype=jnp.int32)])
    pb = jnp.concatenate(
        [perm_b.astype(jnp.int32),
         BATCH + PAD_A + jnp.arange(PAD_B, dtype=jnp.int32)])
    pc = jnp.concatenate(
        [perm_c.astype(jnp.int32),
         BATCH + PAD_A + PAD_B + jnp.arange(PAD_C, dtype=jnp.int32)])

    s = _scatter(ha, hb, hc, pa.reshape(-1, 128), pb.reshape(-1, 128),
                 pc.reshape(-1, 128))
    return _to_canonical(s).T


# speedup vs baseline: 14.5481x; 1.0442x over previous
import functools

import jax
import jax.numpy as jnp
from jax import lax
from jax.experimental import pallas as pl
from jax.experimental.pallas import tpu as pltpu
from jax.experimental.pallas import tpu_sc as plsc

BATCH = 1_000_000
D = 16
HID = 16
OUT = 8
NA, NB, NC = 500_000, 300_000, 200_000

NW = 32
CH = 2048
TPC = CH // 128
CT = 32768

PAD_A = (-NA) % 128
PAD_B = (-NB) % 128
PAD_C = (-NC) % 128
NPA, NPB, NPC = NA + PAD_A, NB + PAD_B, NC + PAD_C
NOUT = ((BATCH + PAD_A + PAD_B + PAD_C + 127) // 128) * 128
OTILES = NOUT // 128


def _mlp_body(zt_ref, w1_ref, b1_ref, w2_ref, b2_ref, out_ref):
    zt = zt_ref[...]
    ht = lax.dot_general(w1_ref[...], zt, (((0,), (0,)), ((), ())),
                         preferred_element_type=jnp.float32)
    ht = jnp.maximum(ht + b1_ref[...][:, 0:1], 0.0)
    ot = lax.dot_general(w2_ref[...], ht, (((0,), (0,)), ((), ())),
                         preferred_element_type=jnp.float32)
    ot = ot + b2_ref[...][:, 0:1]
    out_ref[...] = ot.reshape(OUT, CT // 128, 128).transpose(1, 0, 2)


def _mlp(zt, b1t, b2t, w1, w2, npad):
    n = zt.shape[1]
    grid = (npad + CT - 1) // CT
    return pl.pallas_call(
        _mlp_body,
        grid=(grid,),
        in_specs=[
            pl.BlockSpec((D, CT), lambda i: (0, i)),
            pl.BlockSpec((D, HID), lambda i: (0, 0)),
            pl.BlockSpec((HID, 128), lambda i: (0, 0)),
            pl.BlockSpec((HID, OUT), lambda i: (0, 0)),
            pl.BlockSpec((OUT, 128), lambda i: (0, 0)),
        ],
        out_specs=pl.BlockSpec((CT // 128, OUT, 128), lambda i: (i, 0, 0)),
        out_shape=jax.ShapeDtypeStruct((npad // 128, OUT, 128), jnp.float32),
    )(zt, w1, b1t, w2, b2t)


def _scatter_body(ha, hb, hc, pa, pb, pc, out,
                  idx0, idx1, tiles0, tiles1, rows_v, sst,
                  semld0, semld1, sem_sp, sem_sc):
    s_id = lax.axis_index("s")
    w = s_id * 2 + lax.axis_index("c")

    bufs = ((idx0, tiles0, semld0), (idx1, tiles1, semld1))

    for h_ref, p_ref, n in ((ha, pa, NPA), (hb, pb, NPB), (hc, pc, NPC)):
        nchunks = n // CH if n % CH == 0 else n // CH + 1
        last = n - CH
        niter = (nchunks - w + NW - 1) // NW
        npair = (niter + 1) // 2

        def tbase(i):
            return jnp.minimum((w + i * NW) * CH, last) // 128

        def start_load(i, b, p_ref=p_ref, h_ref=h_ref):
            idx_v, tiles_v, semld = bufs[b]
            tb = tbase(i)
            pltpu.async_copy(p_ref.at[pl.ds(tb, TPC)], idx_v, semld)
            pltpu.async_copy(h_ref.at[pl.ds(tb, TPC)], tiles_v, semld)

        def process(b, p_ref=p_ref, h_ref=h_ref):
            idx_v, tiles_v, semld = bufs[b]
            pltpu.make_async_copy(p_ref.at[pl.ds(0, TPC)], idx_v, semld).wait()
            pltpu.make_async_copy(h_ref.at[pl.ds(0, TPC)], tiles_v,
                                  semld).wait()
            sps = [
                pltpu.async_copy(tiles_v.at[:, j, :], sst.at[s_id, :, :, j],
                                 sem_sp)
                for j in range(OUT)
            ]
            for c in sps:
                c.wait()
            pltpu.sync_copy(sst.at[s_id], rows_v)
            scs = [
                pltpu.async_copy(rows_v.at[t], out.at[idx_v.at[t]], sem_sc)
                for t in range(TPC)
            ]
            for c in scs:
                c.wait()

        @pl.when(niter > 0)
        def _():
            start_load(0, 0)

        def pair(p, _):
            @pl.when(2 * p + 1 < niter)
            def _():
                start_load(2 * p + 1, 1)
            process(0)

            @pl.when(2 * p + 1 < niter)
            def _():
                @pl.when(2 * p + 2 < niter)
                def _():
                    start_load(2 * p + 2, 0)
                process(1)
            return 0

        lax.fori_loop(0, npair, pair, 0)


def _scatter(ha, hb, hc, pa, pb, pc):
    mesh = plsc.VectorSubcoreMesh(core_axis_name="c", subcore_axis_name="s")
    f = functools.partial(
        pl.kernel,
        mesh=mesh,
        compiler_params=pltpu.CompilerParams(use_tc_tiling_on_sc=False),
        out_type=jax.ShapeDtypeStruct((NOUT, OUT), jnp.float32),
        scratch_types=[
            pltpu.VMEM((TPC, 128), jnp.int32),
            pltpu.VMEM((TPC, 128), jnp.int32),
            pltpu.VMEM((TPC, OUT, 128), jnp.float32),
            pltpu.VMEM((TPC, OUT, 128), jnp.float32),
            pltpu.VMEM((TPC, 128, OUT), jnp.float32),
            pltpu.VMEM_SHARED((16, TPC, 128, OUT), jnp.float32),
            pltpu.SemaphoreType.DMA,
            pltpu.SemaphoreType.DMA,
            pltpu.SemaphoreType.DMA,
            pltpu.SemaphoreType.DMA,
        ],
    )(_scatter_body)
    return f(ha, hb, hc, pa, pb, pc)


TB = 64


def _unpack_body(s_ref, out_ref):
    x = s_ref[...]
    y = jnp.transpose(x, (0, 2, 1))
    z = jnp.transpose(y, (1, 0, 2))
    out_ref[...] = z.reshape(OUT, TB * 128)


def _to_canonical(s):
    s3 = s.reshape(OTILES, 128, OUT)
    grid = (BATCH + TB * 128 - 1) // (TB * 128)
    return pl.pallas_call(
        _unpack_body,
        grid=(grid,),
        in_specs=[pl.BlockSpec((TB, 128, OUT), lambda i: (i, 0, 0))],
        out_specs=pl.BlockSpec((OUT, TB * 128), lambda i: (0, i)),
        out_shape=jax.ShapeDtypeStruct((OUT, BATCH), jnp.float32),
    )(s3)


def kernel(z_a, z_b, z_c, W1, b1, W2, b2, perm_a, perm_b, perm_c):
    b1t = jnp.tile(b1[:, None], (1, 128))
    b2t = jnp.tile(b2[:, None], (1, 128))

    ha = _mlp(z_a.T, b1t, b2t, W1, W2, NPA)
    hb = _mlp(z_b.T, b1t, b2t, W1, W2, NPB)
    hc = _mlp(z_c.T, b1t, b2t, W1, W2, NPC)

    pa = jnp.concatenate(
        [perm_a.astype(jnp.int32), BATCH + jnp.arange(PAD_A, dtype=jnp.int32)])
    pb = jnp.concatenate(
        [perm_b.astype(jnp.int32),
         BATCH + PAD_A + jnp.arange(PAD_B, dtype=jnp.int32)])
    pc = jnp.concatenate(
        [perm_c.astype(jnp.int32),
         BATCH + PAD_A + PAD_B + jnp.arange(PAD_C, dtype=jnp.int32)])

    s = _scatter(ha, hb, hc, pa.reshape(-1, 128), pb.reshape(-1, 128),
                 pc.reshape(-1, 128))
    return _to_canonical(s).T


# speedup vs baseline: 15.3101x; 1.0524x over previous
import functools

import jax
import jax.numpy as jnp
from jax import lax
from jax.experimental import pallas as pl
from jax.experimental.pallas import tpu as pltpu
from jax.experimental.pallas import tpu_sc as plsc

BATCH = 1_000_000
D = 16
HID = 16
OUT = 8
NA, NB, NC = 500_000, 300_000, 200_000

NW = 32
CH = 2048
TPC = CH // 128
CT = 32768

PAD_A = (-NA) % 128
PAD_B = (-NB) % 128
PAD_C = (-NC) % 128
NPA, NPB, NPC = NA + PAD_A, NB + PAD_B, NC + PAD_C
NOUT = ((BATCH + PAD_A + PAD_B + PAD_C + 127) // 128) * 128
OTILES = NOUT // 128


def _mlp_body(zt_ref, w1_ref, b1_ref, w2_ref, b2_ref, out_ref):
    zt = zt_ref[...]
    ht = lax.dot_general(w1_ref[...], zt, (((0,), (0,)), ((), ())),
                         preferred_element_type=jnp.float32)
    ht = jnp.maximum(ht + b1_ref[...][:, 0:1], 0.0)
    ot = lax.dot_general(w2_ref[...], ht, (((0,), (0,)), ((), ())),
                         preferred_element_type=jnp.float32)
    ot = ot + b2_ref[...][:, 0:1]
    out_ref[...] = ot.reshape(OUT, CT // 128, 128).transpose(1, 0, 2)


def _mlp(zt, b1t, b2t, w1, w2, npad):
    n = zt.shape[1]
    grid = (npad + CT - 1) // CT
    return pl.pallas_call(
        _mlp_body,
        grid=(grid,),
        in_specs=[
            pl.BlockSpec((D, CT), lambda i: (0, i)),
            pl.BlockSpec((D, HID), lambda i: (0, 0)),
            pl.BlockSpec((HID, 128), lambda i: (0, 0)),
            pl.BlockSpec((HID, OUT), lambda i: (0, 0)),
            pl.BlockSpec((OUT, 128), lambda i: (0, 0)),
        ],
        out_specs=pl.BlockSpec((CT // 128, OUT, 128), lambda i: (i, 0, 0)),
        out_shape=jax.ShapeDtypeStruct((npad // 128, OUT, 128), jnp.float32),
    )(zt, w1, b1t, w2, b2t)


def _scatter_body(ha, hb, hc, pa, pb, pc, out,
                  idx0, idx1, idx2, idx3, tiles0, tiles1, rows0, rows1, sst,
                  semld0, semld1, sem_sp, sem_sc0, sem_sc1):
    s_id = lax.axis_index("s")
    w = s_id * 2 + lax.axis_index("c")

    idxs = (idx0, idx1, idx2, idx3)
    tiles = (tiles0, tiles1)
    rows = (rows0, rows1)
    semlds = (semld0, semld1)
    semscs = (sem_sc0, sem_sc1)

    for h_ref, p_ref, n in ((ha, pa, NPA), (hb, pb, NPB), (hc, pc, NPC)):
        nchunks = n // CH if n % CH == 0 else n // CH + 1
        last = n - CH
        niter = (nchunks - w + NW - 1) // NW
        nquad = (niter + 3) // 4

        def tbase(i):
            return jnp.minimum((w + i * NW) * CH, last) // 128

        def start_load(q, k, p_ref=p_ref, h_ref=h_ref, niter=niter):
            i = 4 * q + k

            @pl.when(i < niter)
            def _():
                tb = tbase(i)
                pltpu.async_copy(p_ref.at[pl.ds(tb, TPC)], idxs[k % 4],
                                 semlds[k % 2])
                pltpu.async_copy(h_ref.at[pl.ds(tb, TPC)], tiles[k % 2],
                                 semlds[k % 2])

        def drain_scat(slot):
            idx_v = idxs[slot % 4]
            rows_v = rows[slot % 2]
            sem_sc = semscs[slot % 2]
            for t in range(TPC):
                pltpu.make_async_copy(rows_v.at[t], out.at[idx_v.at[t]],
                                      sem_sc).wait()

        def process(q, k, p_ref=p_ref, h_ref=h_ref, niter=niter):
            i = 4 * q + k

            @pl.when(i < niter)
            def _():
                idx_v = idxs[k % 4]
                tiles_v = tiles[k % 2]
                rows_v = rows[k % 2]
                semld = semlds[k % 2]
                sem_sc = semscs[k % 2]
                pltpu.make_async_copy(p_ref.at[pl.ds(0, TPC)], idx_v,
                                      semld).wait()
                pltpu.make_async_copy(h_ref.at[pl.ds(0, TPC)], tiles_v,
                                      semld).wait()

                @pl.when(i >= 2)
                def _():
                    drain_scat((k - 2) % 4)

                sps = [
                    pltpu.async_copy(tiles_v.at[:, j, :],
                                     sst.at[s_id, :, :, j], sem_sp)
                    for j in range(OUT)
                ]
                for c in sps:
                    c.wait()
                pltpu.sync_copy(sst.at[s_id], rows_v)
                for t in range(TPC):
                    pltpu.async_copy(rows_v.at[t], out.at[idx_v.at[t]],
                                     sem_sc)

        start_load(0, 0)

        def quad(q, _):
            start_load(q, 1)
            process(q, 0)
            start_load(q, 2)
            process(q, 1)
            start_load(q, 3)
            process(q, 2)
            start_load(q + 1, 0)
            process(q, 3)
            return 0

        lax.fori_loop(0, nquad, quad, 0)

        @pl.when(niter >= 1)
        def _():
            drain_scat(0)

        @pl.when(niter >= 2)
        def _():
            drain_scat(1)


def _scatter(ha, hb, hc, pa, pb, pc):
    mesh = plsc.VectorSubcoreMesh(core_axis_name="c", subcore_axis_name="s")
    f = functools.partial(
        pl.kernel,
        mesh=mesh,
        compiler_params=pltpu.CompilerParams(use_tc_tiling_on_sc=False),
        out_type=jax.ShapeDtypeStruct((NOUT, OUT), jnp.float32),
        scratch_types=[
            pltpu.VMEM((TPC, 128), jnp.int32),
            pltpu.VMEM((TPC, 128), jnp.int32),
            pltpu.VMEM((TPC, 128), jnp.int32),
            pltpu.VMEM((TPC, 128), jnp.int32),
            pltpu.VMEM((TPC, OUT, 128), jnp.float32),
            pltpu.VMEM((TPC, OUT, 128), jnp.float32),
            pltpu.VMEM((TPC, 128, OUT), jnp.float32),
            pltpu.VMEM((TPC, 128, OUT), jnp.float32),
            pltpu.VMEM_SHARED((16, TPC, 128, OUT), jnp.float32),
            pltpu.SemaphoreType.DMA,
            pltpu.SemaphoreType.DMA,
            pltpu.SemaphoreType.DMA,
            pltpu.SemaphoreType.DMA,
            pltpu.SemaphoreType.DMA,
        ],
    )(_scatter_body)
    return f(ha, hb, hc, pa, pb, pc)


TB = 64


def _unpack_body(s_ref, out_ref):
    x = s_ref[...]
    y = jnp.transpose(x, (0, 2, 1))
    z = jnp.transpose(y, (1, 0, 2))
    out_ref[...] = z.reshape(OUT, TB * 128)


def _to_canonical(s):
    s3 = s.reshape(OTILES, 128, OUT)
    grid = (BATCH + TB * 128 - 1) // (TB * 128)
    return pl.pallas_call(
        _unpack_body,
        grid=(grid,),
        in_specs=[pl.BlockSpec((TB, 128, OUT), lambda i: (i, 0, 0))],
        out_specs=pl.BlockSpec((OUT, TB * 128), lambda i: (0, i)),
        out_shape=jax.ShapeDtypeStruct((OUT, BATCH), jnp.float32),
    )(s3)


def kernel(z_a, z_b, z_c, W1, b1, W2, b2, perm_a, perm_b, perm_c):
    b1t = jnp.tile(b1[:, None], (1, 128))
    b2t = jnp.tile(b2[:, None], (1, 128))

    ha = _mlp(z_a.T, b1t, b2t, W1, W2, NPA)
    hb = _mlp(z_b.T, b1t, b2t, W1, W2, NPB)
    hc = _mlp(z_c.T, b1t, b2t, W1, W2, NPC)

    pa = jnp.concatenate(
        [perm_a.astype(jnp.int32), BATCH + jnp.arange(PAD_A, dtype=jnp.int32)])
    pb = jnp.concatenate(
        [perm_b.astype(jnp.int32),
         BATCH + PAD_A + jnp.arange(PAD_B, dtype=jnp.int32)])
    pc = jnp.concatenate(
        [perm_c.astype(jnp.int32),
         BATCH + PAD_A + PAD_B + jnp.arange(PAD_C, dtype=jnp.int32)])

    s = _scatter(ha, hb, hc, pa.reshape(-1, 128), pb.reshape(-1, 128),
                 pc.reshape(-1, 128))
    return _to_canonical(s).T


# speedup vs baseline: 15.4039x; 1.0061x over previous
import functools

import jax
import jax.numpy as jnp
from jax import lax
from jax.experimental import pallas as pl
from jax.experimental.pallas import tpu as pltpu
from jax.experimental.pallas import tpu_sc as plsc

BATCH = 1_000_000
D = 16
HID = 16
OUT = 8
NA, NB, NC = 500_000, 300_000, 200_000

NW = 32
CH = 2048
TPC = CH // 128
CT = 32768

PAD_A = (-NA) % 128
PAD_B = (-NB) % 128
PAD_C = (-NC) % 128
NPA, NPB, NPC = NA + PAD_A, NB + PAD_B, NC + PAD_C
NOUT = ((BATCH + PAD_A + PAD_B + PAD_C + 127) // 128) * 128
OTILES = NOUT // 128


GA = (NPA + CT - 1) // CT
GB = (NPB + CT - 1) // CT
GC = (NPC + CT - 1) // CT


def _mlp3_body(za_ref, zb_ref, zc_ref, w1_ref, b1_ref, w2_ref, b2_ref,
               oa_ref, ob_ref, oc_ref):
    for z_ref, o_ref in ((za_ref, oa_ref), (zb_ref, ob_ref), (zc_ref, oc_ref)):
        zt = z_ref[...]
        ht = lax.dot_general(w1_ref[...], zt, (((0,), (0,)), ((), ())),
                             preferred_element_type=jnp.float32)
        ht = jnp.maximum(ht + b1_ref[...][:, 0:1], 0.0)
        ot = lax.dot_general(w2_ref[...], ht, (((0,), (0,)), ((), ())),
                             preferred_element_type=jnp.float32)
        ot = ot + b2_ref[...][:, 0:1]
        o_ref[...] = ot.reshape(OUT, CT // 128, 128).transpose(1, 0, 2)


def _mlp3(zta, ztb, ztc, b1t, b2t, w1, w2):
    return pl.pallas_call(
        _mlp3_body,
        grid=(GA,),
        in_specs=[
            pl.BlockSpec((D, CT), lambda i: (0, i)),
            pl.BlockSpec((D, CT), lambda i: (0, jnp.minimum(i, GB - 1))),
            pl.BlockSpec((D, CT), lambda i: (0, jnp.minimum(i, GC - 1))),
            pl.BlockSpec((D, HID), lambda i: (0, 0)),
            pl.BlockSpec((HID, 128), lambda i: (0, 0)),
            pl.BlockSpec((HID, OUT), lambda i: (0, 0)),
            pl.BlockSpec((OUT, 128), lambda i: (0, 0)),
        ],
        out_specs=[
            pl.BlockSpec((CT // 128, OUT, 128), lambda i: (i, 0, 0)),
            pl.BlockSpec((CT // 128, OUT, 128),
                         lambda i: (jnp.minimum(i, GB - 1), 0, 0)),
            pl.BlockSpec((CT // 128, OUT, 128),
                         lambda i: (jnp.minimum(i, GC - 1), 0, 0)),
        ],
        out_shape=[
            jax.ShapeDtypeStruct((NPA // 128, OUT, 128), jnp.float32),
            jax.ShapeDtypeStruct((NPB // 128, OUT, 128), jnp.float32),
            jax.ShapeDtypeStruct((NPC // 128, OUT, 128), jnp.float32),
        ],
    )(zta, ztb, ztc, w1, b1t, w2, b2t)


def _scatter_body(ha, hb, hc, pa, pb, pc, out,
                  idx0, idx1, idx2, idx3, tiles0, tiles1, rows0, rows1, sst,
                  semld0, semld1, sem_sp, sem_sc0, sem_sc1):
    s_id = lax.axis_index("s")
    w = s_id * 2 + lax.axis_index("c")

    idxs = (idx0, idx1, idx2, idx3)
    tiles = (tiles0, tiles1)
    rows = (rows0, rows1)
    semlds = (semld0, semld1)
    semscs = (sem_sc0, sem_sc1)

    for h_ref, p_ref, n in ((ha, pa, NPA), (hb, pb, NPB), (hc, pc, NPC)):
        nchunks = n // CH if n % CH == 0 else n // CH + 1
        last = n - CH
        niter = (nchunks - w + NW - 1) // NW
        nquad = (niter + 3) // 4

        def tbase(i):
            return jnp.minimum((w + i * NW) * CH, last) // 128

        def start_load(q, k, p_ref=p_ref, h_ref=h_ref, niter=niter):
            i = 4 * q + k

            @pl.when(i < niter)
            def _():
                tb = tbase(i)
                pltpu.async_copy(p_ref.at[pl.ds(tb, TPC)], idxs[k % 4],
                                 semlds[k % 2])
                pltpu.async_copy(h_ref.at[pl.ds(tb, TPC)], tiles[k % 2],
                                 semlds[k % 2])

        def drain_scat(slot):
            idx_v = idxs[slot % 4]
            rows_v = rows[slot % 2]
            sem_sc = semscs[slot % 2]
            for t in range(TPC):
                pltpu.make_async_copy(rows_v.at[t], out.at[idx_v.at[t]],
                                      sem_sc).wait()

        def process(q, k, p_ref=p_ref, h_ref=h_ref, niter=niter):
            i = 4 * q + k

            @pl.when(i < niter)
            def _():
                idx_v = idxs[k % 4]
                tiles_v = tiles[k % 2]
                rows_v = rows[k % 2]
                semld = semlds[k % 2]
                sem_sc = semscs[k % 2]
                pltpu.make_async_copy(p_ref.at[pl.ds(0, TPC)], idx_v,
                                      semld).wait()
                pltpu.make_async_copy(h_ref.at[pl.ds(0, TPC)], tiles_v,
                                      semld).wait()

                @pl.when(i >= 2)
                def _():
                    drain_scat((k - 2) % 4)

                sps = [
                    pltpu.async_copy(tiles_v.at[:, j, :],
                                     sst.at[s_id, :, :, j], sem_sp)
                    for j in range(OUT)
                ]
                for c in sps:
                    c.wait()
                pltpu.sync_copy(sst.at[s_id], rows_v)
                for t in range(TPC):
                    pltpu.async_copy(rows_v.at[t], out.at[idx_v.at[t]],
                                     sem_sc)

        start_load(0, 0)

        def quad(q, _):
            start_load(q, 1)
            process(q, 0)
            start_load(q, 2)
            process(q, 1)
            start_load(q, 3)
            process(q, 2)
            start_load(q + 1, 0)
            process(q, 3)
            return 0

        lax.fori_loop(0, nquad, quad, 0)

        @pl.when(niter >= 1)
        def _():
            drain_scat(0)

        @pl.when(niter >= 2)
        def _():
            drain_scat(1)


def _scatter(ha, hb, hc, pa, pb, pc):
    mesh = plsc.VectorSubcoreMesh(core_axis_name="c", subcore_axis_name="s")
    f = functools.partial(
        pl.kernel,
        mesh=mesh,
        compiler_params=pltpu.CompilerParams(use_tc_tiling_on_sc=False),
        out_type=jax.ShapeDtypeStruct((NOUT, OUT), jnp.float32),
        scratch_types=[
            pltpu.VMEM((TPC, 128), jnp.int32),
            pltpu.VMEM((TPC, 128), jnp.int32),
            pltpu.VMEM((TPC, 128), jnp.int32),
            pltpu.VMEM((TPC, 128), jnp.int32),
            pltpu.VMEM((TPC, OUT, 128), jnp.float32),
            pltpu.VMEM((TPC, OUT, 128), jnp.float32),
            pltpu.VMEM((TPC, 128, OUT), jnp.float32),
            pltpu.VMEM((TPC, 128, OUT), jnp.float32),
            pltpu.VMEM_SHARED((16, TPC, 128, OUT), jnp.float32),
            pltpu.SemaphoreType.DMA,
            pltpu.SemaphoreType.DMA,
            pltpu.SemaphoreType.DMA,
            pltpu.SemaphoreType.DMA,
            pltpu.SemaphoreType.DMA,
        ],
    )(_scatter_body)
    return f(ha, hb, hc, pa, pb, pc)


TB = 64


def _unpack_body(s_ref, out_ref):
    x = s_ref[...]
    y = jnp.transpose(x, (0, 2, 1))
    z = jnp.transpose(y, (1, 0, 2))
    out_ref[...] = z.reshape(OUT, TB * 128)


def _to_canonical(s):
    s3 = s.reshape(OTILES, 128, OUT)
    grid = (BATCH + TB * 128 - 1) // (TB * 128)
    return pl.pallas_call(
        _unpack_body,
        grid=(grid,),
        in_specs=[pl.BlockSpec((TB, 128, OUT), lambda i: (i, 0, 0))],
        out_specs=pl.BlockSpec((OUT, TB * 128), lambda i: (0, i)),
        out_shape=jax.ShapeDtypeStruct((OUT, BATCH), jnp.float32),
    )(s3)


def kernel(z_a, z_b, z_c, W1, b1, W2, b2, perm_a, perm_b, perm_c):
    b1t = jnp.tile(b1[:, None], (1, 128))
    b2t = jnp.tile(b2[:, None], (1, 128))

    ha, hb, hc = _mlp3(z_a.T, z_b.T, z_c.T, b1t, b2t, W1, W2)

    pa = jnp.concatenate(
        [perm_a.astype(jnp.int32), BATCH + jnp.arange(PAD_A, dtype=jnp.int32)])
    pb = jnp.concatenate(
        [perm_b.astype(jnp.int32),
         BATCH + PAD_A + jnp.arange(PAD_B, dtype=jnp.int32)])
    pc = jnp.concatenate(
        [perm_c.astype(jnp.int32),
         BATCH + PAD_A + PAD_B + jnp.arange(PAD_C, dtype=jnp.int32)])

    s = _scatter(ha, hb, hc, pa.reshape(-1, 128), pb.reshape(-1, 128),
                 pc.reshape(-1, 128))
    return _to_canonical(s).T
